# Initial kernel scaffold; baseline (speedup 1.0000x reference)
import functools

import jax
import jax.numpy as jnp
from jax import lax
from jax.experimental import pallas as pl
from jax.experimental.pallas import tpu as pltpu
from jax.experimental.pallas import tpu_sc as plsc

_VOCAB = 100000
_EMB = 64
_HID = 128
_B = 1024
_L = 200

_NC = 2
_NS = 16
_NW = _NC * _NS
_BPW = _B // _NW

_CHUNKS = ((0, 128), (128, 72))


def _pool_sc(idx_hbm, table_hbm, out_hbm, idx_v, rows_v, pool_v, sem):
    wid = lax.axis_index("s") * _NC + lax.axis_index("c")
    base = wid * _BPW
    pltpu.sync_copy(idx_hbm.at[pl.ds(base, _BPW)], idx_v)

    def row_body(i, carry):
        cps = [
            pltpu.async_copy(
                table_hbm.at[idx_v.at[i, pl.ds(off, n)]],
                rows_v.at[pl.ds(off, n)],
                sem,
            )
            for off, n in _CHUNKS
        ]
        for cp in cps:
            cp.wait()

        def add_r(r, acc):
            return tuple(
                acc[c] + rows_v[r, pl.ds(c * 16, 16)] for c in range(4)
            )

        z = jnp.zeros((16,), jnp.float32)
        acc = lax.fori_loop(0, _L, add_r, (z, z, z, z))
        scale = jnp.float32(1.0 / _L)
        for c in range(4):
            pool_v[pl.ds(c * 16, 16)] = acc[c] * scale
        pltpu.sync_copy(pool_v, out_hbm.at[base + i])
        return carry

    lax.fori_loop(0, _BPW, row_body, 0)


def _pool(inputs, emb_table):
    mesh = plsc.VectorSubcoreMesh(core_axis_name="c", subcore_axis_name="s")
    f = pl.kernel(
        _pool_sc,
        out_type=jax.ShapeDtypeStruct((_B, _EMB), jnp.float32),
        mesh=mesh,
        scratch_types=[
            pltpu.VMEM((_BPW, _L), jnp.int32),
            pltpu.VMEM((_L, _EMB), jnp.float32),
            pltpu.VMEM((_EMB,), jnp.float32),
            pltpu.SemaphoreType.DMA,
        ],
    )
    return f(inputs, emb_table)


def _mlp_tc(pooled_ref, w1_ref, b1_ref, w2_ref, b2_ref, out_ref):
    h = jnp.dot(pooled_ref[...], w1_ref[...],
                preferred_element_type=jnp.float32)
    h = jnp.maximum(h + b1_ref[...], 0.0)
    out_ref[...] = jnp.dot(h, w2_ref[...],
                           preferred_element_type=jnp.float32) + b2_ref[...]


_TN = 2048


def _mlp(pooled, W1, b1, W2, b2):
    nv = pl.cdiv(_VOCAB, _TN)
    return pl.pallas_call(
        _mlp_tc,
        grid=(nv,),
        in_specs=[
            pl.BlockSpec((_B, _EMB), lambda i: (0, 0)),
            pl.BlockSpec((_EMB, _HID), lambda i: (0, 0)),
            pl.BlockSpec((1, _HID), lambda i: (0, 0)),
            pl.BlockSpec((_HID, _TN), lambda i: (0, i)),
            pl.BlockSpec((1, _TN), lambda i: (0, i)),
        ],
        out_specs=pl.BlockSpec((_B, _TN), lambda i: (0, i)),
        out_shape=jax.ShapeDtypeStruct((_B, _VOCAB), jnp.float32),
        compiler_params=pltpu.CompilerParams(
            dimension_semantics=("arbitrary",),
        ),
    )(pooled, W1, b1.reshape(1, _HID), W2, b2.reshape(1, _VOCAB))


def kernel(inputs, emb_table, W1, b1, W2, b2):
    pooled = _pool(inputs, emb_table)
    return _mlp(pooled, W1, b1, W2, b2)


# speedup vs baseline: 1.3926x; 1.3926x over previous
import functools

import jax
import jax.numpy as jnp
from jax import lax
from jax.experimental import pallas as pl
from jax.experimental.pallas import tpu as pltpu
from jax.experimental.pallas import tpu_sc as plsc

_VOCAB = 100000
_EMB = 64
_HID = 128
_B = 1024
_L = 200

_NC = 2
_NS = 16
_NW = _NC * _NS
_BPW = _B // _NW

_CHUNKS = ((0, 128), (128, 72))


def _pool_sc(idx_hbm, table_hbm, out_hbm, idx_v, rows_v, pool_v, sem):
    wid = lax.axis_index("s") * _NC + lax.axis_index("c")
    base = wid * _BPW
    pltpu.sync_copy(idx_hbm.at[pl.ds(base, _BPW)], idx_v)

    def row_body(i, carry):
        cps = [
            pltpu.async_copy(
                table_hbm.at[idx_v.at[i, pl.ds(off, n)]],
                rows_v.at[pl.ds(off, n)],
                sem,
            )
            for off, n in _CHUNKS
        ]
        for cp in cps:
            cp.wait()

        def add_r(r, acc):
            return tuple(
                acc[c] + rows_v[r, pl.ds(c * 16, 16)] for c in range(4)
            )

        z = jnp.zeros((16,), jnp.float32)
        acc = lax.fori_loop(0, _L, add_r, (z, z, z, z))
        scale = jnp.float32(1.0 / _L)
        for c in range(4):
            pool_v[pl.ds(c * 16, 16)] = acc[c] * scale
        pltpu.sync_copy(pool_v, out_hbm.at[base + i])
        return carry

    lax.fori_loop(0, _BPW, row_body, 0)


def _pool(inputs, emb_table):
    mesh = plsc.VectorSubcoreMesh(core_axis_name="c", subcore_axis_name="s")
    f = pl.kernel(
        _pool_sc,
        out_type=jax.ShapeDtypeStruct((_B, _EMB), jnp.float32),
        mesh=mesh,
        scratch_types=[
            pltpu.VMEM((_BPW, _L), jnp.int32),
            pltpu.VMEM((_L, _EMB), jnp.float32),
            pltpu.VMEM((_EMB,), jnp.float32),
            pltpu.SemaphoreType.DMA,
        ],
        compiler_params=pltpu.CompilerParams(use_tc_tiling_on_sc=False),
    )
    return f(inputs, emb_table)


def _mlp_tc(pooled_ref, w1_ref, b1_ref, w2_ref, b2_ref, out_ref):
    h = jnp.dot(pooled_ref[...], w1_ref[...],
                preferred_element_type=jnp.float32)
    h = jnp.maximum(h + b1_ref[...], 0.0)
    out_ref[...] = jnp.dot(h, w2_ref[...],
                           preferred_element_type=jnp.float32) + b2_ref[...]


_TN = 2048


def _mlp(pooled, W1, b1, W2, b2):
    nv = pl.cdiv(_VOCAB, _TN)
    return pl.pallas_call(
        _mlp_tc,
        grid=(nv,),
        in_specs=[
            pl.BlockSpec((_B, _EMB), lambda i: (0, 0)),
            pl.BlockSpec((_EMB, _HID), lambda i: (0, 0)),
            pl.BlockSpec((1, _HID), lambda i: (0, 0)),
            pl.BlockSpec((_HID, _TN), lambda i: (0, i)),
            pl.BlockSpec((1, _TN), lambda i: (0, i)),
        ],
        out_specs=pl.BlockSpec((_B, _TN), lambda i: (0, i)),
        out_shape=jax.ShapeDtypeStruct((_B, _VOCAB), jnp.float32),
        compiler_params=pltpu.CompilerParams(
            dimension_semantics=("arbitrary",),
        ),
    )(pooled, W1, b1.reshape(1, _HID), W2, b2.reshape(1, _VOCAB))


def kernel(inputs, emb_table, W1, b1, W2, b2):
    pooled = _pool(inputs, emb_table)
    return _mlp(pooled, W1, b1, W2, b2)


# speedup vs baseline: 1.4284x; 1.0257x over previous
import functools

import jax
import jax.numpy as jnp
from jax import lax
from jax.experimental import pallas as pl
from jax.experimental.pallas import tpu as pltpu
from jax.experimental.pallas import tpu_sc as plsc

_VOCAB = 100000
_EMB = 64
_HID = 128
_B = 1024
_L = 200

_NC = 2
_NS = 16
_NW = _NC * _NS
_BPW = _B // _NW

_G = 4
_NG = _BPW // _G
_GI = _G * _L
_GCHUNKS = tuple((o, min(128, _GI - o)) for o in range(0, _GI, 128))


def _issue_group(table_hbm, idx_v, buf, base, sem):
    for off, n in _GCHUNKS:
        pltpu.async_copy(
            table_hbm.at[idx_v.at[pl.ds(base + off, n)]],
            buf.at[pl.ds(off, n)],
            sem,
        )


def _drain_group(table_hbm, idx_v, buf, base, sem):
    for off, n in _GCHUNKS:
        pltpu.make_async_copy(
            table_hbm.at[idx_v.at[pl.ds(base + off, n)]],
            buf.at[pl.ds(off, n)],
            sem,
        ).wait()


def _accum_group(buf, pool_v, out_hbm, wbase, g):
    scale = jnp.float32(1.0 / _L)
    for r in range(_G):
        def add_r(k, acc):
            return tuple(
                acc[c] + buf[r * _L + k, pl.ds(c * 16, 16)] for c in range(4)
            )
        z = jnp.zeros((16,), jnp.float32)
        acc = lax.fori_loop(0, _L, add_r, (z, z, z, z))
        for c in range(4):
            pool_v[r, pl.ds(c * 16, 16)] = acc[c] * scale
    pltpu.sync_copy(pool_v, out_hbm.at[pl.ds(wbase + g * _G, _G)])


def _pool_sc(idx_hbm, table_hbm, out_hbm, idx_v, buf_a, buf_b, pool_v,
             sem_a, sem_b):
    wid = lax.axis_index("s") * _NC + lax.axis_index("c")
    wbase = wid * _BPW
    pltpu.sync_copy(idx_hbm.at[pl.ds(wbase * _L, _BPW * _L)], idx_v)

    _issue_group(table_hbm, idx_v, buf_a, 0, sem_a)

    def pair_body(p, carry):
        g0 = p * 2
        i0 = pl.multiple_of(g0 * _GI, 8)

        @pl.when(g0 + 1 < _NG)
        def _():
            _issue_group(table_hbm, idx_v, buf_b, i0 + _GI, sem_b)
        _drain_group(table_hbm, idx_v, buf_a, i0, sem_a)
        _accum_group(buf_a, pool_v, out_hbm, wbase, g0)

        @pl.when(g0 + 2 < _NG)
        def _():
            _issue_group(table_hbm, idx_v, buf_a, i0 + 2 * _GI, sem_a)
        _drain_group(table_hbm, idx_v, buf_b, i0 + _GI, sem_b)
        _accum_group(buf_b, pool_v, out_hbm, wbase, g0 + 1)
        return carry

    lax.fori_loop(0, _NG // 2, pair_body, 0)


def _pool(inputs, emb_table):
    mesh = plsc.VectorSubcoreMesh(core_axis_name="c", subcore_axis_name="s")
    f = pl.kernel(
        _pool_sc,
        out_type=jax.ShapeDtypeStruct((_B, _EMB), jnp.float32),
        mesh=mesh,
        scratch_types=[
            pltpu.VMEM((_BPW * _L,), jnp.int32),
            pltpu.VMEM((_GI, _EMB), jnp.float32),
            pltpu.VMEM((_GI, _EMB), jnp.float32),
            pltpu.VMEM((_G, _EMB), jnp.float32),
            pltpu.SemaphoreType.DMA,
            pltpu.SemaphoreType.DMA,
        ],
        compiler_params=pltpu.CompilerParams(use_tc_tiling_on_sc=False),
    )
    return f(inputs.reshape(_B * _L), emb_table)


def _mlp_tc(pooled_ref, w1_ref, b1_ref, w2_ref, b2_ref, out_ref):
    h = jnp.dot(pooled_ref[...], w1_ref[...],
                preferred_element_type=jnp.float32)
    h = jnp.maximum(h + b1_ref[...], 0.0)
    out_ref[...] = jnp.dot(h, w2_ref[...],
                           preferred_element_type=jnp.float32) + b2_ref[...]


_TN = 4096


def _mlp(pooled, W1, b1, W2, b2):
    nv = pl.cdiv(_VOCAB, _TN)
    return pl.pallas_call(
        _mlp_tc,
        grid=(nv,),
        in_specs=[
            pl.BlockSpec((_B, _EMB), lambda i: (0, 0)),
            pl.BlockSpec((_EMB, _HID), lambda i: (0, 0)),
            pl.BlockSpec((1, _HID), lambda i: (0, 0)),
            pl.BlockSpec((_HID, _TN), lambda i: (0, i)),
            pl.BlockSpec((1, _TN), lambda i: (0, i)),
        ],
        out_specs=pl.BlockSpec((_B, _TN), lambda i: (0, i)),
        out_shape=jax.ShapeDtypeStruct((_B, _VOCAB), jnp.float32),
        compiler_params=pltpu.CompilerParams(
            dimension_semantics=("arbitrary",),
        ),
    )(pooled, W1, b1.reshape(1, _HID), W2, b2.reshape(1, _VOCAB))


def kernel(inputs, emb_table, W1, b1, W2, b2):
    pooled = _pool(inputs, emb_table)
    return _mlp(pooled, W1, b1, W2, b2)


# speedup vs baseline: 1.4301x; 1.0012x over previous
import functools

import jax
import jax.numpy as jnp
from jax import lax
from jax.experimental import pallas as pl
from jax.experimental.pallas import tpu as pltpu
from jax.experimental.pallas import tpu_sc as plsc

_VOCAB = 100000
_EMB = 64
_HID = 128
_B = 1024
_L = 200

_NC = 2
_NS = 16
_NW = _NC * _NS
_BPW = _B // _NW

_G = 4
_NG = _BPW // _G
_GI = _G * _L
_GCHUNKS = tuple((o, min(128, _GI - o)) for o in range(0, _GI, 128))


def _issue_group(table_hbm, idx_v, buf, base, sem):
    for off, n in _GCHUNKS:
        pltpu.async_copy(
            table_hbm.at[idx_v.at[pl.ds(base + off, n)]],
            buf.at[pl.ds(off, n)],
            sem,
        )


def _drain_group(table_hbm, idx_v, buf, base, sem):
    for off, n in _GCHUNKS:
        pltpu.make_async_copy(
            table_hbm.at[idx_v.at[pl.ds(base + off, n)]],
            buf.at[pl.ds(off, n)],
            sem,
        ).wait()


def _accum_group(buf, pool_v, out_hbm, wbase, g):
    scale = jnp.float32(1.0 / _L)
    for r in range(_G):
        def add_r(k, acc):
            return tuple(
                acc[c] + buf[r * _L + k, pl.ds(c * 16, 16)] for c in range(4)
            )
        z = jnp.zeros((16,), jnp.float32)
        acc = lax.fori_loop(0, _L, add_r, (z, z, z, z))
        for c in range(4):
            pool_v[r, pl.ds(c * 16, 16)] = acc[c] * scale
    pltpu.sync_copy(pool_v, out_hbm.at[pl.ds(wbase + g * _G, _G)])


def _pool_sc(idx_hbm, table_hbm, out_hbm, idx_v, buf_a, buf_b, pool_v,
             sem_a, sem_b):
    wid = lax.axis_index("s") * _NC + lax.axis_index("c")
    wbase = wid * _BPW
    pltpu.sync_copy(idx_hbm.at[pl.ds(wbase * _L, _BPW * _L)], idx_v)

    _issue_group(table_hbm, idx_v, buf_a, 0, sem_a)

    def pair_body(p, carry):
        g0 = p * 2
        i0 = pl.multiple_of(g0 * _GI, 8)

        @pl.when(g0 + 1 < _NG)
        def _():
            _issue_group(table_hbm, idx_v, buf_b, i0 + _GI, sem_b)
        _drain_group(table_hbm, idx_v, buf_a, i0, sem_a)
        _accum_group(buf_a, pool_v, out_hbm, wbase, g0)

        @pl.when(g0 + 2 < _NG)
        def _():
            _issue_group(table_hbm, idx_v, buf_a, i0 + 2 * _GI, sem_a)
        _drain_group(table_hbm, idx_v, buf_b, i0 + _GI, sem_b)
        _accum_group(buf_b, pool_v, out_hbm, wbase, g0 + 1)
        return carry

    lax.fori_loop(0, _NG // 2, pair_body, 0)


def _pool(inputs, emb_table):
    mesh = plsc.VectorSubcoreMesh(core_axis_name="c", subcore_axis_name="s")
    f = pl.kernel(
        _pool_sc,
        out_type=jax.ShapeDtypeStruct((_B, _EMB), jnp.float32),
        mesh=mesh,
        scratch_types=[
            pltpu.VMEM((_BPW * _L,), jnp.int32),
            pltpu.VMEM((_GI, _EMB), jnp.float32),
            pltpu.VMEM((_GI, _EMB), jnp.float32),
            pltpu.VMEM((_G, _EMB), jnp.float32),
            pltpu.SemaphoreType.DMA,
            pltpu.SemaphoreType.DMA,
        ],
        compiler_params=pltpu.CompilerParams(use_tc_tiling_on_sc=False),
    )
    idx_flat = lax.optimization_barrier(inputs.reshape(_B * _L))
    return f(idx_flat, emb_table)


def _mlp_tc(pooled_ref, w1_ref, b1_ref, w2_ref, b2_ref, out_ref):
    h = jnp.dot(pooled_ref[...], w1_ref[...],
                preferred_element_type=jnp.float32)
    h = jnp.maximum(h + b1_ref[...], 0.0)
    out_ref[...] = jnp.dot(h, w2_ref[...],
                           preferred_element_type=jnp.float32) + b2_ref[...]


_TN = 4096


def _mlp(pooled, W1, b1, W2, b2):
    nv = pl.cdiv(_VOCAB, _TN)
    return pl.pallas_call(
        _mlp_tc,
        grid=(nv,),
        in_specs=[
            pl.BlockSpec((_B, _EMB), lambda i: (0, 0)),
            pl.BlockSpec((_EMB, _HID), lambda i: (0, 0)),
            pl.BlockSpec((1, _HID), lambda i: (0, 0)),
            pl.BlockSpec((_HID, _TN), lambda i: (0, i)),
            pl.BlockSpec((1, _TN), lambda i: (0, i)),
        ],
        out_specs=pl.BlockSpec((_B, _TN), lambda i: (0, i)),
        out_shape=jax.ShapeDtypeStruct((_B, _VOCAB), jnp.float32),
        compiler_params=pltpu.CompilerParams(
            dimension_semantics=("arbitrary",),
        ),
    )(pooled, W1, b1.reshape(1, _HID), W2, b2.reshape(1, _VOCAB))


def kernel(inputs, emb_table, W1, b1, W2, b2):
    pooled = _pool(inputs, emb_table)
    return _mlp(pooled, W1, b1, W2, b2)
